# baseline (speedup 1.0000x reference)
import functools

import jax
import jax.numpy as jnp
from jax import lax
from jax.experimental import pallas as pl
from jax.experimental.pallas import tpu as pltpu
from jax.experimental.pallas import tpu_sc as plsc

_MAX_LENGTH = 512
_D = 768
_B = 64
_L = 512

_NW = 32
_C = 32
_TPW = _B * _L // _NW
_NCH = _TPW // _C

_mesh = plsc.VectorSubcoreMesh(core_axis_name="c", subcore_axis_name="s")


def _posid_body(m_ref, g_ref):
    m = m_ref[...]
    valid = (1 - m).astype(jnp.float32)
    r = lax.broadcasted_iota(jnp.int32, (_L, _L), 0)
    c = lax.broadcasted_iota(jnp.int32, (_L, _L), 1)
    tri = (r <= c).astype(jnp.float32)
    cs = jnp.dot(valid, tri, preferred_element_type=jnp.float32)
    pos = cs.astype(jnp.int32) - 1
    g_ref[...] = jnp.where(m != 0, _MAX_LENGTH, pos)


_posid = pl.pallas_call(
    _posid_body,
    out_shape=jax.ShapeDtypeStruct((_B, _L), jnp.int32),
)


@functools.partial(
    pl.kernel,
    mesh=_mesh,
    out_type=[
        jax.ShapeDtypeStruct((_B * _L, _D), jnp.float32),
        jax.ShapeDtypeStruct((_B * _L, _D), jnp.float32),
    ],
    scratch_types=[
        pltpu.VMEM((_NCH, _C), jnp.int32),
        pltpu.VMEM((_NCH, _C), jnp.int32),
        pltpu.VMEM((2, _C, _D), jnp.float32),
        pltpu.VMEM((2, _C, _D), jnp.float32),
        pltpu.SemaphoreType.DMA,
        pltpu.SemaphoreType.DMA,
        pltpu.SemaphoreType.DMA,
        pltpu.SemaphoreType.DMA,
    ],
)
def _sc_gather(ids_hbm, g_hbm, tok_hbm, pose_hbm, x_hbm, xe_hbm,
               idx_v, g_v, tok_b, pos_b, sin0, sin1, sout0, sout1):
    wid = lax.axis_index("s") * 2 + lax.axis_index("c")

    pltpu.sync_copy(ids_hbm.at[wid], idx_v)
    pltpu.sync_copy(g_hbm.at[wid], g_v)

    base = wid * _TPW
    sem_in = (sin0, sin1)
    sem_out = (sout0, sout1)
    in_h = [None, None]
    out_h = [None, None]

    def start_in(c):
        b = c % 2
        in_h[b] = (
            pltpu.async_copy(tok_hbm.at[idx_v.at[c]], tok_b.at[b], sem_in[b]),
            pltpu.async_copy(pose_hbm.at[g_v.at[c]], pos_b.at[b], sem_in[b]),
        )

    def start_out(c):
        b = c % 2
        off = base + c * _C
        out_h[b] = (
            pltpu.async_copy(tok_b.at[b], x_hbm.at[pl.ds(off, _C)], sem_out[b]),
            pltpu.async_copy(pos_b.at[b], xe_hbm.at[pl.ds(off, _C)], sem_out[b]),
        )

    start_in(0)
    for c in range(_NCH):
        b = c % 2
        if c + 1 < _NCH:
            if c >= 1:
                for h in out_h[(c + 1) % 2]:
                    h.wait()
            start_in(c + 1)
        for h in in_h[b]:
            h.wait()
        start_out(c)
    for pair in out_h:
        for h in pair:
            h.wait()


def _pos_ext(mod_vec):
    arange = jnp.arange(_MAX_LENGTH, dtype=jnp.float32)
    pos_dim = _D // 2
    omega = jnp.arange(pos_dim, dtype=jnp.float32) / pos_dim
    omega = 1.0 / (10000.0 ** omega)
    out = arange[:, None] * omega[None, :]
    pos = jnp.concatenate([jnp.sin(out), jnp.cos(out)], axis=1)
    return jnp.concatenate([pos + mod_vec[None, :], mod_vec[None, :]], axis=0)


def kernel(tensor, input_mask, token_emb, mod_emb):
    g = _posid(input_mask.astype(jnp.int32))
    ids = tensor.astype(jnp.int32).reshape(_NW, _NCH, _C)
    g3 = g.reshape(_NW, _NCH, _C)
    pose = _pos_ext(mod_emb.reshape(_D))
    x, xe = _sc_gather(ids, g3, token_emb, pose)
    return x.reshape(_B, _L, _D), xe.reshape(_B, _L, _D)


# speedup vs baseline: 6.6839x; 6.6839x over previous
import functools

import jax
import jax.numpy as jnp
from jax import lax
from jax.experimental import pallas as pl
from jax.experimental.pallas import tpu as pltpu
from jax.experimental.pallas import tpu_sc as plsc

_MAX_LENGTH = 512
_D = 768
_B = 64
_L = 512
_PE = 520

_NW = 32
_C = 64
_TPW = _B * _L // _NW
_NCH = _TPW // _C

_mesh = plsc.VectorSubcoreMesh(core_axis_name="c", subcore_axis_name="s")


@functools.partial(
    pl.kernel,
    mesh=_mesh,
    out_type=[jax.ShapeDtypeStruct((_B * _L, _D), jnp.float32)],
    scratch_types=[
        pltpu.VMEM((_NCH, _C), jnp.int32),
        pltpu.VMEM((2, _C, _D), jnp.float32),
        pltpu.SemaphoreType.DMA,
        pltpu.SemaphoreType.DMA,
        pltpu.SemaphoreType.DMA,
        pltpu.SemaphoreType.DMA,
    ],
)
def _sc_gather(ids_hbm, tok_hbm, x_hbm, idx_v, row_b, sin0, sin1, sout0, sout1):
    wid = lax.axis_index("s") * 2 + lax.axis_index("c")
    pltpu.sync_copy(ids_hbm.at[wid], idx_v)

    base = wid * _TPW
    sem_in = (sin0, sin1)
    sem_out = (sout0, sout1)
    in_h = [None, None]
    out_h = [None, None]

    def start_in(c):
        b = c % 2
        in_h[b] = pltpu.async_copy(tok_hbm.at[idx_v.at[c]], row_b.at[b], sem_in[b])

    def start_out(c):
        b = c % 2
        off = base + c * _C
        out_h[b] = pltpu.async_copy(row_b.at[b], x_hbm.at[pl.ds(off, _C)], sem_out[b])

    start_in(0)
    for c in range(_NCH):
        b = c % 2
        if c + 1 < _NCH:
            if c >= 1:
                out_h[(c + 1) % 2].wait()
            start_in(c + 1)
        in_h[b].wait()
        start_out(c)
    for h in out_h:
        h.wait()


def _xe_body(m_ref, hi_ref, lo_ref, xe_ref):
    m = m_ref[0]
    valid = (1 - m).astype(jnp.bfloat16)
    r = lax.broadcasted_iota(jnp.int32, (_L, _L), 0)
    c = lax.broadcasted_iota(jnp.int32, (_L, _L), 1)
    tri = (r <= c).astype(jnp.bfloat16)
    cs = jnp.dot(valid, tri, preferred_element_type=jnp.float32)
    pos = cs.astype(jnp.int32) - 1
    g = jnp.where(m != 0, _MAX_LENGTH, pos)
    prow = lax.broadcasted_iota(jnp.int32, (_PE, _L), 0)
    oht = (prow == g).astype(jnp.bfloat16)
    acc = lax.dot_general(oht, hi_ref[...], (((0,), (0,)), ((), ())),
                          preferred_element_type=jnp.float32)
    acc = acc + lax.dot_general(oht, lo_ref[...], (((0,), (0,)), ((), ())),
                                preferred_element_type=jnp.float32)
    xe_ref[...] = acc[None]


_xe = pl.pallas_call(
    _xe_body,
    grid=(_B,),
    in_specs=[
        pl.BlockSpec((1, 1, _L), lambda i: (i, 0, 0)),
        pl.BlockSpec((_PE, _D), lambda i: (0, 0)),
        pl.BlockSpec((_PE, _D), lambda i: (0, 0)),
    ],
    out_specs=pl.BlockSpec((1, _L, _D), lambda i: (i, 0, 0)),
    out_shape=jax.ShapeDtypeStruct((_B, _L, _D), jnp.float32),
)


def _pos_ext(mod_vec):
    arange = jnp.arange(_MAX_LENGTH, dtype=jnp.float32)
    pos_dim = _D // 2
    omega = jnp.arange(pos_dim, dtype=jnp.float32) / pos_dim
    omega = 1.0 / (10000.0 ** omega)
    out = arange[:, None] * omega[None, :]
    pos = jnp.concatenate([jnp.sin(out), jnp.cos(out)], axis=1)
    full = jnp.concatenate([pos + mod_vec[None, :], mod_vec[None, :]], axis=0)
    return jnp.pad(full, ((0, _PE - _MAX_LENGTH - 1), (0, 0)))


def kernel(tensor, input_mask, token_emb, mod_emb):
    ids = tensor.astype(jnp.int32).reshape(_NW, _NCH, _C)
    pose = _pos_ext(mod_emb.reshape(_D))
    hi = pose.astype(jnp.bfloat16)
    lo = (pose - hi.astype(jnp.float32)).astype(jnp.bfloat16)
    (x,) = _sc_gather(ids, token_emb)
    xe = _xe(input_mask.astype(jnp.int32).reshape(_B, 1, _L), hi, lo)
    return x.reshape(_B, _L, _D), xe
